# baseline (speedup 1.0000x reference)
import jax
import jax.numpy as jnp
from jax import lax
from jax.experimental import pallas as pl
from jax.experimental.pallas import tpu as pltpu
from jax.experimental.pallas import tpu_sc as plsc

IOU_EPS = 1e-7
EPS3 = IOU_EPS / 3.0
ONE_THIRD = 1.0 / 3.0

_NC = 2
_NS = 16
_NW = _NC * _NS
_L = 16

_N_PAD = 5120
_NP = _N_PAD // _NW
_T = 8
_M = 2000
_MR = 2016
_MC2 = _MR // (2 * _L)
_UNROLL = 2
_CAP = _M + _UNROLL * _L
_PCAP = _NP + _L


def _sc_body(pred_hbm, dp_hbm, out_hbm,
             dpra, dprb, sema, semb, cmp_v, predv,
             cpx1, cpy1, cpx2, cpy2, cpa3, cidx, outv):
    wid = lax.axis_index("s") * _NC + lax.axis_index("c")
    base = wid * _NP
    iota = lax.iota(jnp.int32, _L)
    lane0 = iota == 0

    pltpu.sync_copy(pred_hbm.at[pl.ds(wid * 4 * _NP, 4 * _NP)], predv)

    def _zero(i, carry):
        outv[pl.ds(i * _L, _L)] = jnp.zeros((_L,), jnp.float32)
        return carry

    lax.fori_loop(0, _NP // _L, _zero, 0)

    dprs = (dpra, dprb)
    sems = (sema, semb)
    handles = [None, None]
    handles[0] = pltpu.async_copy(dp_hbm.at[0], dprs[0], sems[0])
    cnt = []
    for t in range(_T):
        cur = t % 2
        if t + 1 < _T:
            handles[1 - cur] = pltpu.async_copy(dp_hbm.at[t + 1],
                                                dprs[1 - cur], sems[1 - cur])
        handles[cur].wait()
        dpr = dprs[cur]

        def _compact(j, off, t=t, dpr=dpr):
            for u in range(2):
                jb = j * (2 * _L) + u * _L
                x1 = dpr[pl.ds(jb, _L)]
                y1 = dpr[pl.ds(_MR + jb, _L)]
                x2 = dpr[pl.ds(2 * _MR + jb, _L)]
                y2 = dpr[pl.ds(3 * _MR + jb, _L)]
                msk = (x2 > x1) & (y2 > y1)
                thr3 = ((x2 - x1) * (y2 - y1) + IOU_EPS) * ONE_THIRD
                mi = msk.astype(jnp.int32)
                csum = plsc.cumsum(mi)
                rb = jnp.full((_L,), t * 5 * _CAP + off, jnp.int32)
                dst = (csum - mi) + rb
                plsc.store_scatter(cmp_v, [dst], x1, mask=msk)
                cap1 = jnp.full((_L,), _CAP, jnp.int32)
                plsc.store_scatter(cmp_v, [dst + cap1], y1, mask=msk)
                plsc.store_scatter(cmp_v, [dst + cap1 + cap1], x2, mask=msk)
                plsc.store_scatter(
                    cmp_v, [dst + jnp.full((_L,), 3 * _CAP, jnp.int32)],
                    y2, mask=msk)
                plsc.store_scatter(
                    cmp_v, [dst + jnp.full((_L,), 4 * _CAP, jnp.int32)],
                    thr3, mask=msk)
                off = off + jnp.max(csum)
            return off

        c_t = lax.fori_loop(0, _MC2, _compact, jnp.int32(0))
        zer = jnp.zeros((_L,), jnp.float32)
        for c in range(5):
            for u in range(_UNROLL):
                cmp_v[pl.ds(t * 5 * _CAP + c * _CAP + c_t + u * _L, _L)] = zer
        cnt.append(c_t)

    def _pcompact(i, off):
        ib = i * _L
        x1 = predv[pl.ds(ib, _L)]
        y1 = predv[pl.ds(_NP + ib, _L)]
        x2 = predv[pl.ds(2 * _NP + ib, _L)]
        y2 = predv[pl.ds(3 * _NP + ib, _L)]
        msk = (x2 > x1) & (y2 > y1)
        pa3 = ((x2 - x1) * (y2 - y1)) * ONE_THIRD
        lid = iota + jnp.full((_L,), ib, jnp.int32)
        mi = msk.astype(jnp.int32)
        csum = plsc.cumsum(mi)
        dst = (csum - mi) + jnp.full((_L,), off, jnp.int32)
        plsc.store_scatter(cpx1, [dst], x1, mask=msk)
        plsc.store_scatter(cpy1, [dst], y1, mask=msk)
        plsc.store_scatter(cpx2, [dst], x2, mask=msk)
        plsc.store_scatter(cpy2, [dst], y2, mask=msk)
        plsc.store_scatter(cpa3, [dst], pa3, mask=msk)
        plsc.store_scatter(cidx, [dst], lid, mask=msk)
        return off + jnp.max(csum)

    pcnt = lax.fori_loop(0, _NP // _L, _pcompact, jnp.int32(0))

    npair = (pcnt + 1) // 2

    def _per_pair(p, carry):
        k0 = p * 2
        k1 = jnp.minimum(k0 + 1, pcnt - 1)
        ks0 = jnp.full((_L,), k0, jnp.int32)
        ks1 = jnp.full((_L,), k1, jnp.int32)
        ax1 = plsc.load_gather(cpx1, [ks0])
        ay1 = plsc.load_gather(cpy1, [ks0])
        ax2 = plsc.load_gather(cpx2, [ks0])
        ay2 = plsc.load_gather(cpy2, [ks0])
        aa3 = plsc.load_gather(cpa3, [ks0])
        qx1 = plsc.load_gather(cpx1, [ks1])
        qy1 = plsc.load_gather(cpy1, [ks1])
        qx2 = plsc.load_gather(cpx2, [ks1])
        qy2 = plsc.load_gather(cpy2, [ks1])
        qa3 = plsc.load_gather(cpa3, [ks1])
        tot0 = jnp.float32(0.0)
        tot1 = jnp.float32(0.0)
        for t in range(_T):
            nch2 = (cnt[t] + (_UNROLL * _L - 1)) // (_UNROLL * _L)

            def _chunk(j, fnd, t=t, ax1=ax1, ay1=ay1, ax2=ax2, ay2=ay2,
                       aa3=aa3, qx1=qx1, qy1=qy1, qx2=qx2, qy2=qy2, qa3=qa3):
                f0, f1 = fnd
                rb = t * 5 * _CAP + j * (_UNROLL * _L)
                m0 = m1 = None
                for u in range(_UNROLL):
                    ub = rb + u * _L
                    bx1 = cmp_v[pl.ds(ub, _L)]
                    by1 = cmp_v[pl.ds(_CAP + ub, _L)]
                    bx2 = cmp_v[pl.ds(2 * _CAP + ub, _L)]
                    by2 = cmp_v[pl.ds(3 * _CAP + ub, _L)]
                    bt3 = cmp_v[pl.ds(4 * _CAP + ub, _L)]
                    w0 = jnp.maximum(
                        jnp.minimum(ax2, bx2) - jnp.maximum(ax1, bx1), 0.0)
                    h0 = jnp.maximum(
                        jnp.minimum(ay2, by2) - jnp.maximum(ay1, by1), 0.0)
                    mu0 = w0 * h0 > aa3 + bt3
                    m0 = mu0 if m0 is None else m0 | mu0
                    w1 = jnp.maximum(
                        jnp.minimum(qx2, bx2) - jnp.maximum(qx1, bx1), 0.0)
                    h1 = jnp.maximum(
                        jnp.minimum(qy2, by2) - jnp.maximum(qy1, by1), 0.0)
                    mu1 = w1 * h1 > qa3 + bt3
                    m1 = mu1 if m1 is None else m1 | mu1
                return f0 | jnp.any(m0), f1 | jnp.any(m1)

            f0, f1 = lax.fori_loop(0, nch2, _chunk,
                                   (jnp.bool_(False), jnp.bool_(False)))
            tot0 = tot0 + jnp.where(f0, jnp.float32(1.0 / _T), jnp.float32(0.0))
            tot1 = tot1 + jnp.where(f1, jnp.float32(1.0 / _T), jnp.float32(0.0))
        lid0 = plsc.load_gather(cidx, [ks0])
        lid1 = plsc.load_gather(cidx, [ks1])
        plsc.store_scatter(outv, [lid1], jnp.full((_L,), tot1, jnp.float32),
                           mask=lane0)
        plsc.store_scatter(outv, [lid0], jnp.full((_L,), tot0, jnp.float32),
                           mask=lane0)
        return carry

    lax.fori_loop(0, npair, _per_pair, 0)

    pltpu.sync_copy(outv, out_hbm.at[pl.ds(base, _NP)])


def kernel(pred, dropout_preds, dropout_cls_confs):
    del dropout_cls_confs
    N = pred.shape[0]

    pred_c = (jnp.zeros((4, _N_PAD), jnp.float32)
              .at[:, :N].set(pred[:, :4].T)
              .reshape(4, _NP, _NW).transpose(2, 0, 1).reshape(-1))
    dp_c = (jnp.zeros((_T, 4, _MR), jnp.float32)
            .at[:, :, :_M].set(dropout_preds[:, :, :4].transpose(0, 2, 1))
            .reshape(_T, 4 * _MR))

    mesh = plsc.VectorSubcoreMesh(core_axis_name="c", subcore_axis_name="s")
    run = pl.kernel(
        _sc_body,
        mesh=mesh,
        out_type=jax.ShapeDtypeStruct((_N_PAD,), jnp.float32),
        compiler_params=pltpu.CompilerParams(needs_layout_passes=False),
        scratch_types=[
            pltpu.VMEM((4 * _MR,), jnp.float32),
            pltpu.VMEM((4 * _MR,), jnp.float32),
            pltpu.SemaphoreType.DMA,
            pltpu.SemaphoreType.DMA,
            pltpu.VMEM((_T * 5 * _CAP,), jnp.float32),
            pltpu.VMEM((4 * _NP,), jnp.float32),
            pltpu.VMEM((_PCAP,), jnp.float32),
            pltpu.VMEM((_PCAP,), jnp.float32),
            pltpu.VMEM((_PCAP,), jnp.float32),
            pltpu.VMEM((_PCAP,), jnp.float32),
            pltpu.VMEM((_PCAP,), jnp.float32),
            pltpu.VMEM((_PCAP,), jnp.int32),
            pltpu.VMEM((_NP,), jnp.float32),
        ],
    )
    out = run(pred_c, dp_c)
    return out.reshape(_NW, _NP).T.reshape(-1)[:N]


# speedup vs baseline: 1.0168x; 1.0168x over previous
import jax
import jax.numpy as jnp
from jax import lax
from jax.experimental import pallas as pl
from jax.experimental.pallas import tpu as pltpu
from jax.experimental.pallas import tpu_sc as plsc

IOU_EPS = 1e-7
EPS3 = IOU_EPS / 3.0
ONE_THIRD = 1.0 / 3.0

_NC = 2
_NS = 16
_NW = _NC * _NS
_L = 16

_N_PAD = 5120
_NP = _N_PAD // _NW
_T = 8
_M = 2000
_MC = _M // _L
_UNROLL = 2
_CAP = _M + _UNROLL * _L
_PCAP = _NP + _L


def _sc_body(pred_hbm, dp_hbm, out_hbm,
             dpra, dprb, sema, semb, cmp_v, predv,
             cpx1, cpy1, cpx2, cpy2, cpa3, cidx, outv):
    wid = lax.axis_index("s") * _NC + lax.axis_index("c")
    base = wid * _NP
    iota = lax.iota(jnp.int32, _L)
    lane0 = iota == 0

    pltpu.sync_copy(pred_hbm.at[pl.ds(wid * 4 * _NP, 4 * _NP)], predv)

    def _zero(i, carry):
        outv[pl.ds(i * _L, _L)] = jnp.zeros((_L,), jnp.float32)
        return carry

    lax.fori_loop(0, _NP // _L, _zero, 0)

    dprs = (dpra, dprb)
    sems = (sema, semb)
    handles = [None, None]
    handles[0] = pltpu.async_copy(dp_hbm.at[0], dprs[0], sems[0])
    cnt = []
    for t in range(_T):
        cur = t % 2
        if t + 1 < _T:
            handles[1 - cur] = pltpu.async_copy(dp_hbm.at[t + 1],
                                                dprs[1 - cur], sems[1 - cur])
        handles[cur].wait()
        dpr = dprs[cur]

        def _compact(j, off, t=t, dpr=dpr):
            jb = j * _L
            x1 = dpr[pl.ds(jb, _L)]
            y1 = dpr[pl.ds(_M + jb, _L)]
            x2 = dpr[pl.ds(2 * _M + jb, _L)]
            y2 = dpr[pl.ds(3 * _M + jb, _L)]
            msk = (x2 > x1) & (y2 > y1)
            thr3 = ((x2 - x1) * (y2 - y1) + IOU_EPS) * ONE_THIRD
            mi = msk.astype(jnp.int32)
            csum = plsc.cumsum(mi)
            rb = jnp.full((_L,), t * 5 * _CAP + off, jnp.int32)
            dst = (csum - mi) + rb
            plsc.store_scatter(cmp_v, [dst], x1, mask=msk)
            cap1 = jnp.full((_L,), _CAP, jnp.int32)
            plsc.store_scatter(cmp_v, [dst + cap1], y1, mask=msk)
            plsc.store_scatter(cmp_v, [dst + cap1 + cap1], x2, mask=msk)
            plsc.store_scatter(
                cmp_v, [dst + jnp.full((_L,), 3 * _CAP, jnp.int32)],
                y2, mask=msk)
            plsc.store_scatter(
                cmp_v, [dst + jnp.full((_L,), 4 * _CAP, jnp.int32)],
                thr3, mask=msk)
            return off + jnp.max(csum)

        c_t = lax.fori_loop(0, _MC, _compact, jnp.int32(0))
        zer = jnp.zeros((_L,), jnp.float32)
        for c in range(5):
            for u in range(_UNROLL):
                cmp_v[pl.ds(t * 5 * _CAP + c * _CAP + c_t + u * _L, _L)] = zer
        cnt.append(c_t)

    def _pcompact(i, off):
        ib = i * _L
        x1 = predv[pl.ds(ib, _L)]
        y1 = predv[pl.ds(_NP + ib, _L)]
        x2 = predv[pl.ds(2 * _NP + ib, _L)]
        y2 = predv[pl.ds(3 * _NP + ib, _L)]
        msk = (x2 > x1) & (y2 > y1)
        pa3 = ((x2 - x1) * (y2 - y1)) * ONE_THIRD
        lid = iota + jnp.full((_L,), ib, jnp.int32)
        mi = msk.astype(jnp.int32)
        csum = plsc.cumsum(mi)
        dst = (csum - mi) + jnp.full((_L,), off, jnp.int32)
        plsc.store_scatter(cpx1, [dst], x1, mask=msk)
        plsc.store_scatter(cpy1, [dst], y1, mask=msk)
        plsc.store_scatter(cpx2, [dst], x2, mask=msk)
        plsc.store_scatter(cpy2, [dst], y2, mask=msk)
        plsc.store_scatter(cpa3, [dst], pa3, mask=msk)
        plsc.store_scatter(cidx, [dst], lid, mask=msk)
        return off + jnp.max(csum)

    pcnt = lax.fori_loop(0, _NP // _L, _pcompact, jnp.int32(0))

    npair = (pcnt + 1) // 2

    def _per_pair(p, carry):
        k0 = p * 2
        k1 = jnp.minimum(k0 + 1, pcnt - 1)
        ks0 = jnp.full((_L,), k0, jnp.int32)
        ks1 = jnp.full((_L,), k1, jnp.int32)
        ax1 = plsc.load_gather(cpx1, [ks0])
        ay1 = plsc.load_gather(cpy1, [ks0])
        ax2 = plsc.load_gather(cpx2, [ks0])
        ay2 = plsc.load_gather(cpy2, [ks0])
        aa3 = plsc.load_gather(cpa3, [ks0])
        qx1 = plsc.load_gather(cpx1, [ks1])
        qy1 = plsc.load_gather(cpy1, [ks1])
        qx2 = plsc.load_gather(cpx2, [ks1])
        qy2 = plsc.load_gather(cpy2, [ks1])
        qa3 = plsc.load_gather(cpa3, [ks1])
        tot0 = jnp.float32(0.0)
        tot1 = jnp.float32(0.0)
        for t in range(_T):
            nch2 = (cnt[t] + (_UNROLL * _L - 1)) // (_UNROLL * _L)

            def _chunk(j, fnd, t=t, ax1=ax1, ay1=ay1, ax2=ax2, ay2=ay2,
                       aa3=aa3, qx1=qx1, qy1=qy1, qx2=qx2, qy2=qy2, qa3=qa3):
                f0, f1 = fnd
                rb = t * 5 * _CAP + j * (_UNROLL * _L)
                m0 = m1 = None
                for u in range(_UNROLL):
                    ub = rb + u * _L
                    bx1 = cmp_v[pl.ds(ub, _L)]
                    by1 = cmp_v[pl.ds(_CAP + ub, _L)]
                    bx2 = cmp_v[pl.ds(2 * _CAP + ub, _L)]
                    by2 = cmp_v[pl.ds(3 * _CAP + ub, _L)]
                    bt3 = cmp_v[pl.ds(4 * _CAP + ub, _L)]
                    w0 = jnp.maximum(
                        jnp.minimum(ax2, bx2) - jnp.maximum(ax1, bx1), 0.0)
                    h0 = jnp.maximum(
                        jnp.minimum(ay2, by2) - jnp.maximum(ay1, by1), 0.0)
                    mu0 = w0 * h0 > aa3 + bt3
                    m0 = mu0 if m0 is None else m0 | mu0
                    w1 = jnp.maximum(
                        jnp.minimum(qx2, bx2) - jnp.maximum(qx1, bx1), 0.0)
                    h1 = jnp.maximum(
                        jnp.minimum(qy2, by2) - jnp.maximum(qy1, by1), 0.0)
                    mu1 = w1 * h1 > qa3 + bt3
                    m1 = mu1 if m1 is None else m1 | mu1
                return f0 | jnp.any(m0), f1 | jnp.any(m1)

            f0, f1 = lax.fori_loop(0, nch2, _chunk,
                                   (jnp.bool_(False), jnp.bool_(False)))
            tot0 = tot0 + jnp.where(f0, jnp.float32(1.0 / _T), jnp.float32(0.0))
            tot1 = tot1 + jnp.where(f1, jnp.float32(1.0 / _T), jnp.float32(0.0))
        lid0 = plsc.load_gather(cidx, [ks0])
        lid1 = plsc.load_gather(cidx, [ks1])
        plsc.store_scatter(outv, [lid1], jnp.full((_L,), tot1, jnp.float32),
                           mask=lane0)
        plsc.store_scatter(outv, [lid0], jnp.full((_L,), tot0, jnp.float32),
                           mask=lane0)
        return carry

    lax.fori_loop(0, npair, _per_pair, 0)

    pltpu.sync_copy(outv, out_hbm.at[pl.ds(base, _NP)])


def kernel(pred, dropout_preds, dropout_cls_confs):
    del dropout_cls_confs
    N = pred.shape[0]

    pred_c = (jnp.zeros((4, _N_PAD), jnp.float32)
              .at[:, :N].set(pred[:, :4].T)
              .reshape(4, _NP, _NW).transpose(2, 0, 1).reshape(-1))
    dp_c = dropout_preds[:, :, :4].transpose(0, 2, 1).reshape(_T, 4 * _M)

    mesh = plsc.VectorSubcoreMesh(core_axis_name="c", subcore_axis_name="s")
    run = pl.kernel(
        _sc_body,
        mesh=mesh,
        out_type=jax.ShapeDtypeStruct((_N_PAD,), jnp.float32),
        compiler_params=pltpu.CompilerParams(needs_layout_passes=False),
        scratch_types=[
            pltpu.VMEM((4 * _M,), jnp.float32),
            pltpu.VMEM((4 * _M,), jnp.float32),
            pltpu.SemaphoreType.DMA,
            pltpu.SemaphoreType.DMA,
            pltpu.VMEM((_T * 5 * _CAP,), jnp.float32),
            pltpu.VMEM((4 * _NP,), jnp.float32),
            pltpu.VMEM((_PCAP,), jnp.float32),
            pltpu.VMEM((_PCAP,), jnp.float32),
            pltpu.VMEM((_PCAP,), jnp.float32),
            pltpu.VMEM((_PCAP,), jnp.float32),
            pltpu.VMEM((_PCAP,), jnp.float32),
            pltpu.VMEM((_PCAP,), jnp.int32),
            pltpu.VMEM((_NP,), jnp.float32),
        ],
    )
    out = run(pred_c, dp_c)
    return out.reshape(_NW, _NP).T.reshape(-1)[:N]


# speedup vs baseline: 1.0661x; 1.0484x over previous
import jax
import jax.numpy as jnp
from jax import lax
from jax.experimental import pallas as pl
from jax.experimental.pallas import tpu as pltpu
from jax.experimental.pallas import tpu_sc as plsc

IOU_EPS = 1e-7
ONE_THIRD = 1.0 / 3.0

_NC = 2
_NS = 16
_NW = _NC * _NS
_L = 16

_N_PAD = 5120
_NP = _N_PAD // _NW
_T = 8
_M = 2000
_MC = _M // _L
_UNROLL = 3
_CAP = _M + _UNROLL * _L
_PCAP = _NP + _L


def _sc_body(pred_hbm, dp_hbm, out_hbm,
             dpra, dprb, sema, semb, cmp_v, predv,
             cpx1, cpy1, cpx2, cpy2, cpa3, cidx, outv):
    wid = lax.axis_index("s") * _NC + lax.axis_index("c")
    base = wid * _NP
    iota = lax.iota(jnp.int32, _L)
    lane0 = iota == 0

    pltpu.sync_copy(pred_hbm.at[pl.ds(wid * 4 * _NP, 4 * _NP)], predv)

    def _zero(i, carry):
        outv[pl.ds(i * _L, _L)] = jnp.zeros((_L,), jnp.float32)
        return carry

    lax.fori_loop(0, _NP // _L, _zero, 0)

    dprs = (dpra, dprb)
    sems = (sema, semb)
    handles = [None, None]
    handles[0] = pltpu.async_copy(dp_hbm.at[0], dprs[0], sems[0])
    cnt = []
    for t in range(_T):
        cur = t % 2
        if t + 1 < _T:
            handles[1 - cur] = pltpu.async_copy(dp_hbm.at[t + 1],
                                                dprs[1 - cur], sems[1 - cur])
        handles[cur].wait()
        dpr = dprs[cur]

        def _compact(j, off, t=t, dpr=dpr):
            jb = j * _L
            x1 = dpr[pl.ds(jb, _L)]
            y1 = dpr[pl.ds(_M + jb, _L)]
            x2 = dpr[pl.ds(2 * _M + jb, _L)]
            y2 = dpr[pl.ds(3 * _M + jb, _L)]
            msk = (x2 > x1) & (y2 > y1)
            thr3 = ((x2 - x1) * (y2 - y1) + IOU_EPS) * ONE_THIRD
            mi = msk.astype(jnp.int32)
            csum = plsc.cumsum(mi)
            rb = jnp.full((_L,), t * 5 * _CAP + off, jnp.int32)
            dst = (csum - mi) + rb
            plsc.store_scatter(cmp_v, [dst], x1, mask=msk)
            cap1 = jnp.full((_L,), _CAP, jnp.int32)
            plsc.store_scatter(cmp_v, [dst + cap1], y1, mask=msk)
            plsc.store_scatter(cmp_v, [dst + cap1 + cap1], x2, mask=msk)
            plsc.store_scatter(
                cmp_v, [dst + jnp.full((_L,), 3 * _CAP, jnp.int32)],
                y2, mask=msk)
            plsc.store_scatter(
                cmp_v, [dst + jnp.full((_L,), 4 * _CAP, jnp.int32)],
                thr3, mask=msk)
            return off + jnp.max(csum)

        c_t = lax.fori_loop(0, _MC, _compact, jnp.int32(0))
        zer = jnp.zeros((_L,), jnp.float32)
        for c in range(5):
            for u in range(_UNROLL):
                cmp_v[pl.ds(t * 5 * _CAP + c * _CAP + c_t + u * _L, _L)] = zer
        cnt.append(c_t)

    def _pcompact(i, off):
        ib = i * _L
        x1 = predv[pl.ds(ib, _L)]
        y1 = predv[pl.ds(_NP + ib, _L)]
        x2 = predv[pl.ds(2 * _NP + ib, _L)]
        y2 = predv[pl.ds(3 * _NP + ib, _L)]
        msk = (x2 > x1) & (y2 > y1)
        pa3 = ((x2 - x1) * (y2 - y1)) * ONE_THIRD
        lid = iota + jnp.full((_L,), ib, jnp.int32)
        mi = msk.astype(jnp.int32)
        csum = plsc.cumsum(mi)
        dst = (csum - mi) + jnp.full((_L,), off, jnp.int32)
        plsc.store_scatter(cpx1, [dst], x1, mask=msk)
        plsc.store_scatter(cpy1, [dst], y1, mask=msk)
        plsc.store_scatter(cpx2, [dst], x2, mask=msk)
        plsc.store_scatter(cpy2, [dst], y2, mask=msk)
        plsc.store_scatter(cpa3, [dst], pa3, mask=msk)
        plsc.store_scatter(cidx, [dst], lid, mask=msk)
        return off + jnp.max(csum)

    pcnt = lax.fori_loop(0, _NP // _L, _pcompact, jnp.int32(0))

    npair = (pcnt + 1) // 2

    def _per_pair(p, carry):
        k0 = p * 2
        k1 = jnp.minimum(k0 + 1, pcnt - 1)
        ks0 = jnp.full((_L,), k0, jnp.int32)
        ks1 = jnp.full((_L,), k1, jnp.int32)
        ax1 = plsc.load_gather(cpx1, [ks0])
        ay1 = plsc.load_gather(cpy1, [ks0])
        ax2 = plsc.load_gather(cpx2, [ks0])
        ay2 = plsc.load_gather(cpy2, [ks0])
        aa3 = plsc.load_gather(cpa3, [ks0])
        qx1 = plsc.load_gather(cpx1, [ks1])
        qy1 = plsc.load_gather(cpy1, [ks1])
        qx2 = plsc.load_gather(cpx2, [ks1])
        qy2 = plsc.load_gather(cpy2, [ks1])
        qa3 = plsc.load_gather(cpa3, [ks1])
        tot0 = jnp.float32(0.0)
        tot1 = jnp.float32(0.0)
        for t in range(_T):
            nch2 = (cnt[t] + (_UNROLL * _L - 1)) // (_UNROLL * _L)

            def _chunk(j, fnd, t=t, ax1=ax1, ay1=ay1, ax2=ax2, ay2=ay2,
                       aa3=aa3, qx1=qx1, qy1=qy1, qx2=qx2, qy2=qy2, qa3=qa3):
                f0, f1 = fnd
                rb = t * 5 * _CAP + j * (_UNROLL * _L)
                m0 = m1 = None
                for u in range(_UNROLL):
                    ub = rb + u * _L
                    bx1 = cmp_v[pl.ds(ub, _L)]
                    by1 = cmp_v[pl.ds(_CAP + ub, _L)]
                    bx2 = cmp_v[pl.ds(2 * _CAP + ub, _L)]
                    by2 = cmp_v[pl.ds(3 * _CAP + ub, _L)]
                    bt3 = cmp_v[pl.ds(4 * _CAP + ub, _L)]
                    w0 = jnp.maximum(
                        jnp.minimum(ax2, bx2) - jnp.maximum(ax1, bx1), 0.0)
                    h0 = jnp.maximum(
                        jnp.minimum(ay2, by2) - jnp.maximum(ay1, by1), 0.0)
                    mu0 = w0 * h0 > aa3 + bt3
                    m0 = mu0 if m0 is None else m0 | mu0
                    w1 = jnp.maximum(
                        jnp.minimum(qx2, bx2) - jnp.maximum(qx1, bx1), 0.0)
                    h1 = jnp.maximum(
                        jnp.minimum(qy2, by2) - jnp.maximum(qy1, by1), 0.0)
                    mu1 = w1 * h1 > qa3 + bt3
                    m1 = mu1 if m1 is None else m1 | mu1
                return f0 | jnp.any(m0), f1 | jnp.any(m1)

            f0, f1 = lax.fori_loop(0, nch2, _chunk,
                                   (jnp.bool_(False), jnp.bool_(False)))
            tot0 = tot0 + jnp.where(f0, jnp.float32(1.0 / _T), jnp.float32(0.0))
            tot1 = tot1 + jnp.where(f1, jnp.float32(1.0 / _T), jnp.float32(0.0))
        lid0 = plsc.load_gather(cidx, [ks0])
        lid1 = plsc.load_gather(cidx, [ks1])
        plsc.store_scatter(outv, [lid1], jnp.full((_L,), tot1, jnp.float32),
                           mask=lane0)
        plsc.store_scatter(outv, [lid0], jnp.full((_L,), tot0, jnp.float32),
                           mask=lane0)
        return carry

    lax.fori_loop(0, npair, _per_pair, 0)

    pltpu.sync_copy(outv, out_hbm.at[pl.ds(base, _NP)])


def kernel(pred, dropout_preds, dropout_cls_confs):
    del dropout_cls_confs
    N = pred.shape[0]

    pred_c = (jnp.zeros((4, _N_PAD), jnp.float32)
              .at[:, :N].set(pred[:, :4].T)
              .reshape(4, _NP, _NW).transpose(2, 0, 1).reshape(-1))
    dp_c = dropout_preds[:, :, :4].transpose(0, 2, 1).reshape(_T, 4 * _M)

    mesh = plsc.VectorSubcoreMesh(core_axis_name="c", subcore_axis_name="s")
    run = pl.kernel(
        _sc_body,
        mesh=mesh,
        out_type=jax.ShapeDtypeStruct((_N_PAD,), jnp.float32),
        compiler_params=pltpu.CompilerParams(needs_layout_passes=False),
        scratch_types=[
            pltpu.VMEM((4 * _M,), jnp.float32),
            pltpu.VMEM((4 * _M,), jnp.float32),
            pltpu.SemaphoreType.DMA,
            pltpu.SemaphoreType.DMA,
            pltpu.VMEM((_T * 5 * _CAP,), jnp.float32),
            pltpu.VMEM((4 * _NP,), jnp.float32),
            pltpu.VMEM((_PCAP,), jnp.float32),
            pltpu.VMEM((_PCAP,), jnp.float32),
            pltpu.VMEM((_PCAP,), jnp.float32),
            pltpu.VMEM((_PCAP,), jnp.float32),
            pltpu.VMEM((_PCAP,), jnp.float32),
            pltpu.VMEM((_PCAP,), jnp.int32),
            pltpu.VMEM((_NP,), jnp.float32),
        ],
    )
    out = run(pred_c, dp_c)
    return out.reshape(_NW, _NP).T.reshape(-1)[:N]
